# Initial kernel scaffold; baseline (speedup 1.0000x reference)
import jax
import jax.numpy as jnp
from jax.experimental import pallas as pl


def kernel(pts, grid0, grid1, grid2, grid3):
    raise NotImplementedError("write your pallas kernel here")


# speedup vs baseline: 3.7331x; 3.7331x over previous
import functools

import jax
import jax.numpy as jnp
from jax import lax
from jax.experimental import pallas as pl
from jax.experimental.pallas import tpu as pltpu
from jax.experimental.pallas import tpu_sc as plsc

_LODS = (16, 32, 64, 128)
_FEAT = 8
_NPTS = 200000
_NC = 2
_NS = 16
_NW = _NC * _NS
_B = 16
_CHUNK = 6256
_NPAD = _NW * _CHUNK


def _interp_kernel(pts_hbm, g0, g1, g2, g3, out_hbm,
                   pts_v, idx0, idx1, idx2, idx3, buf0, buf1, buf2, buf3,
                   out_blk, sem0, sem1, sem2, sem3):
    grids = (g0, g1, g2, g3)
    idx_v = (idx0, idx1, idx2, idx3)
    buf_v = (buf0, buf1, buf2, buf3)
    sems = (sem0, sem1, sem2, sem3)
    cid = lax.axis_index("c")
    sid = lax.axis_index("s")
    wid = sid * _NC + cid
    base = wid * _CHUNK
    npts_w = jnp.minimum(jnp.int32(_CHUNK), jnp.int32(_NPTS) - base)
    nb = lax.shift_right_logical(npts_w, 4)

    pltpu.sync_copy(pts_hbm.at[pl.ds(base * 4, _CHUNK * 4)], pts_v)

    iota = lax.iota(jnp.int32, _B)

    def batch_body(b, carry):
        row0 = b * _B
        flat0 = (row0 + iota) * 4
        x = plsc.load_gather(pts_v, [flat0])
        y = plsc.load_gather(pts_v, [flat0 + 1])
        z = plsc.load_gather(pts_v, [flat0 + 2])
        x = jnp.minimum(jnp.maximum(x, 0.0), 1.0)
        y = jnp.minimum(jnp.maximum(y, 0.0), 1.0)
        z = jnp.minimum(jnp.maximum(z, 0.0), 1.0)

        fracs = []
        copies = []
        for l, res in enumerate(_LODS):
            scale = jnp.float32(res - 1)
            cx = x * scale
            cy = y * scale
            cz = z * scale
            ix = jnp.minimum(cx.astype(jnp.int32), res - 2)
            iy = jnp.minimum(cy.astype(jnp.int32), res - 2)
            iz = jnp.minimum(cz.astype(jnp.int32), res - 2)
            fracs.append((cx - ix.astype(jnp.float32),
                          cy - iy.astype(jnp.float32),
                          cz - iz.astype(jnp.float32)))
            f000 = (ix * res + iy) * res + iz
            offs = (0, 1, res, res + 1,
                    res * res, res * res + 1, res * res + res, res * res + res + 1)
            for c, off in enumerate(offs):
                idx_v[l][pl.ds(c * _B, _B)] = f000 + off
            copies.append(pltpu.async_copy(grids[l].at[idx_v[l]], buf_v[l], sems[l]))

        for l in range(len(_LODS)):
            fx, fy, fz = fracs[l]
            gx = 1.0 - fx
            gy = 1.0 - fy
            gz = 1.0 - fz
            u00 = gx * gy
            u01 = gx * fy
            u10 = fx * gy
            u11 = fx * fy
            w = (u00 * gz, u00 * fz, u01 * gz, u01 * fz,
                 u10 * gz, u10 * fz, u11 * gz, u11 * fz)
            copies[l].wait()
            for f in range(_FEAT):
                col = jnp.full((_B,), f, jnp.int32)
                acc = w[0] * plsc.load_gather(buf_v[l], [iota, col])
                for c in range(1, 8):
                    v = plsc.load_gather(buf_v[l], [iota + c * _B, col])
                    acc = acc + w[c] * v
                plsc.store_scatter(
                    out_blk, [iota, jnp.full((_B,), l * _FEAT + f, jnp.int32)], acc)

        pltpu.sync_copy(out_blk, out_hbm.at[pl.ds(base + row0, _B)])
        return carry

    lax.fori_loop(0, nb, batch_body, jnp.int32(0))


def kernel(pts, grid0, grid1, grid2, grid3):
    pts4 = jnp.pad(pts, ((0, _NPAD - _NPTS), (0, 1))).reshape(-1)
    mesh = plsc.VectorSubcoreMesh(core_axis_name="c", subcore_axis_name="s")
    k = functools.partial(
        pl.kernel,
        mesh=mesh,
        out_type=jax.ShapeDtypeStruct((_NPTS, 4 * _FEAT), jnp.float32),
        compiler_params=pltpu.CompilerParams(
            needs_layout_passes=False, use_tc_tiling_on_sc=False),
        scratch_types=[
            pltpu.VMEM((_CHUNK * 4,), jnp.float32),
            pltpu.VMEM((8 * _B,), jnp.int32),
            pltpu.VMEM((8 * _B,), jnp.int32),
            pltpu.VMEM((8 * _B,), jnp.int32),
            pltpu.VMEM((8 * _B,), jnp.int32),
            pltpu.VMEM((8 * _B, _FEAT), jnp.float32),
            pltpu.VMEM((8 * _B, _FEAT), jnp.float32),
            pltpu.VMEM((8 * _B, _FEAT), jnp.float32),
            pltpu.VMEM((8 * _B, _FEAT), jnp.float32),
            pltpu.VMEM((_B, 4 * _FEAT), jnp.float32),
            pltpu.SemaphoreType.DMA,
            pltpu.SemaphoreType.DMA,
            pltpu.SemaphoreType.DMA,
            pltpu.SemaphoreType.DMA,
        ],
    )(_interp_kernel)
    return k(pts4, grid0, grid1, grid2, grid3)


# speedup vs baseline: 5.0490x; 1.3525x over previous
import functools

import jax
import jax.numpy as jnp
from jax import lax
from jax.experimental import pallas as pl
from jax.experimental.pallas import tpu as pltpu
from jax.experimental.pallas import tpu_sc as plsc

_LODS = (16, 32, 64, 128)
_FEAT = 8
_NPTS = 200000
_NC = 2
_NS = 16
_NW = _NC * _NS
_B = 16
_GRP = 4
_OUTW = 4 * _FEAT
_CHUNK = 6272
_NPAD = _NW * _CHUNK


def _interp_kernel(pts_hbm, g0, g1, g2, g3, out_hbm,
                   xv, yv, zv, idx_v, buf_v, oblk, gsem, osem):
    grids = (g0, g1, g2, g3)
    cid = lax.axis_index("c")
    sid = lax.axis_index("s")
    wid = sid * _NC + cid
    base = wid * _CHUNK
    npts_w = jnp.minimum(jnp.int32(_CHUNK), jnp.int32(_NPTS) - base)
    nb = lax.shift_right_logical(npts_w, 4)
    ng = lax.shift_right_logical(npts_w, 6)

    pltpu.sync_copy(pts_hbm.at[0, pl.ds(base, _CHUNK)], xv)
    pltpu.sync_copy(pts_hbm.at[1, pl.ds(base, _CHUNK)], yv)
    pltpu.sync_copy(pts_hbm.at[2, pl.ds(base, _CHUNK)], zv)

    iota = lax.iota(jnp.int32, _B)

    def coords(b):
        row0 = b * _B
        x = xv[pl.ds(row0, _B)]
        y = yv[pl.ds(row0, _B)]
        z = zv[pl.ds(row0, _B)]
        x = jnp.minimum(jnp.maximum(x, 0.0), 1.0)
        y = jnp.minimum(jnp.maximum(y, 0.0), 1.0)
        z = jnp.minimum(jnp.maximum(z, 0.0), 1.0)
        return x, y, z

    def lod_setup(x, y, z, res):
        scale = jnp.float32(res - 1)
        cx = x * scale
        cy = y * scale
        cz = z * scale
        ix = jnp.minimum(cx.astype(jnp.int32), res - 2)
        iy = jnp.minimum(cy.astype(jnp.int32), res - 2)
        iz = jnp.minimum(cz.astype(jnp.int32), res - 2)
        f000 = (ix * res + iy) * res + iz
        return (cx - ix.astype(jnp.float32), cy - iy.astype(jnp.float32),
                cz - iz.astype(jnp.float32), f000)

    def fire(b, slot):
        x, y, z = coords(b)
        for l, res in enumerate(_LODS):
            _, _, _, f000 = lod_setup(x, y, z, res)
            offs = (0, 1, res, res + 1,
                    res * res, res * res + 1, res * res + res, res * res + res + 1)
            for c, off in enumerate(offs):
                idx_v[slot, l, pl.ds(c * _B, _B)] = f000 + off
            pltpu.async_copy(grids[l].at[idx_v.at[slot, l]],
                             buf_v.at[pl.ds((slot * 4 + l) * (8 * _B), 8 * _B)],
                             gsem.at[slot])

    def wait_gathers(slot):
        for l in range(4):
            pltpu.make_async_copy(
                grids[l].at[idx_v.at[slot, l]],
                buf_v.at[pl.ds((slot * 4 + l) * (8 * _B), 8 * _B)],
                gsem.at[slot]).wait()

    def oflush_copy(g):
        p = lax.rem(g, 2)
        return pltpu.make_async_copy(
            oblk.at[pl.ds(p * (_GRP * _B), _GRP * _B)],
            out_hbm.at[pl.ds(base + g * (_GRP * _B), _GRP * _B)],
            osem.at[p])

    fire(jnp.int32(0), jnp.int32(0))

    def batch_body(b, carry):
        slot = lax.rem(b, 2)
        g = lax.shift_right_logical(b, 2)
        bo = lax.rem(b, _GRP)

        @pl.when(b + 1 < nb)
        def _prefetch():
            fire(b + 1, 1 - slot)

        @pl.when(jnp.logical_and(bo == 0, g >= 2))
        def _drain_out():
            oflush_copy(g - 2).wait()

        wait_gathers(slot)

        x, y, z = coords(b)
        orow0 = lax.rem(b, 2 * _GRP) * _B
        for l, res in enumerate(_LODS):
            fx, fy, fz, _ = lod_setup(x, y, z, res)
            gx = 1.0 - fx
            gy = 1.0 - fy
            gz = 1.0 - fz
            u00 = gx * gy
            u01 = gx * fy
            u10 = fx * gy
            u11 = fx * fy
            w = (u00 * gz, u00 * fz, u01 * gz, u01 * fz,
                 u10 * gz, u10 * fz, u11 * gz, u11 * fz)
            rb = (slot * 4 + l) * (8 * _B) + iota
            for f in range(_FEAT):
                col = jnp.full((_B,), f, jnp.int32)
                acc = w[0] * plsc.load_gather(buf_v, [rb, col])
                for c in range(1, 8):
                    v = plsc.load_gather(buf_v, [rb + c * _B, col])
                    acc = acc + w[c] * v
                plsc.store_scatter(
                    oblk, [orow0 + iota, jnp.full((_B,), l * _FEAT + f, jnp.int32)],
                    acc)

        @pl.when(bo == _GRP - 1)
        def _flush():
            oflush_copy(g).start()

        return carry

    lax.fori_loop(0, nb, batch_body, jnp.int32(0))

    oflush_copy(ng - 2).wait()
    oflush_copy(ng - 1).wait()


def kernel(pts, grid0, grid1, grid2, grid3):
    ptst = jnp.pad(pts.T, ((0, 0), (0, _NPAD - _NPTS)))
    mesh = plsc.VectorSubcoreMesh(core_axis_name="c", subcore_axis_name="s")
    k = functools.partial(
        pl.kernel,
        mesh=mesh,
        out_type=jax.ShapeDtypeStruct((_NPTS, _OUTW), jnp.float32),
        compiler_params=pltpu.CompilerParams(
            needs_layout_passes=False, use_tc_tiling_on_sc=False),
        scratch_types=(
            [pltpu.VMEM((_CHUNK,), jnp.float32) for _ in range(3)]
            + [pltpu.VMEM((2, 4, 8 * _B), jnp.int32),
               pltpu.VMEM((2 * 4 * 8 * _B, _FEAT), jnp.float32),
               pltpu.VMEM((2 * _GRP * _B, _OUTW), jnp.float32),
               pltpu.SemaphoreType.DMA((2,)),
               pltpu.SemaphoreType.DMA((2,))]
        ),
    )(_interp_kernel)
    return k(ptst, grid0, grid1, grid2, grid3)


# speedup vs baseline: 17.1054x; 3.3879x over previous
import functools

import jax
import jax.numpy as jnp
from jax import lax
from jax.experimental import pallas as pl
from jax.experimental.pallas import tpu as pltpu
from jax.experimental.pallas import tpu_sc as plsc

_LODS = (16, 32, 64, 128)
_FEAT = 8
_NPTS = 200000
_NC = 2
_NS = 16
_NW = _NC * _NS
_B = 16
_GRP = 4
_NSLOT = 4
_OUTW = 4 * _FEAT
_GOFF = (0, 16 ** 3, 16 ** 3 + 32 ** 3, 16 ** 3 + 32 ** 3 + 64 ** 3)
_TROWS = 16 ** 3 + 32 ** 3 + 64 ** 3 + 128 ** 3
_NBLK = tuple(r ** 3 // 128 for r in _LODS)
_KB = (2, 8, 8, 8)
_CHUNK = 6272
_NPAD = _NW * _CHUNK
_NCB = (_NPTS + 127) // 128


def _interp_kernel(pts_hbm, g0, g1, g2, g3, out_hbm, tab_hbm,
                   xv, yv, zv, idx_v, buf_v, oblk, nbuf, tbuf, g0tile,
                   gsem, osem, nsem, tsem):
    grids = (g0, g1, g2, g3)
    cid = lax.axis_index("c")
    sid = lax.axis_index("s")
    wid = sid * _NC + cid
    iota = lax.iota(jnp.int32, _B)

    tp_row = iota >> 3
    tp_col = iota & 7

    def g0_in(ci, start):
        cp = pltpu.make_async_copy(
            g0.at[pl.ds(ci * 64, 64)],
            nbuf.at[pl.ds(lax.rem(ci, 2) * 64, 64)],
            nsem.at[lax.rem(ci, 2)])
        if start:
            cp.start()
        else:
            cp.wait()

    g0_in(jnp.int32(0), True)

    def g0_body(ci, carry):
        s = lax.rem(ci, 2)

        @pl.when(ci + 1 < 4)
        def _pref():
            g0_in(ci + 1, True)

        g0_in(ci, False)
        nrow0 = tp_col + s * 64
        trow0v = tp_row + ci * 1024

        @plsc.parallel_loop(0, 512, 1, unroll=16)
        def _relane0(k):
            rowv = nrow0 + lax.shift_right_logical(k, 6) * _FEAT
            colv = tp_row + 2 * lax.rem(k, 64)
            v = plsc.load_gather(nbuf, [rowv, colv])
            plsc.store_scatter(g0tile, [trow0v + 2 * k, tp_col], v)

        return carry

    lax.fori_loop(0, 4, g0_body, jnp.int32(0))

    for l in range(1, 4):
        nbt = _NBLK[l] // _NS
        kb = _KB[l]
        ni = nbt // kb
        blk0 = sid * nbt
        grow0 = _GOFF[l]

        def p0_in(ci, start, l=l, blk0=blk0, kb=kb):
            cp = pltpu.make_async_copy(
                grids[l].at[pl.ds((blk0 + ci * kb) * _FEAT, kb * _FEAT)],
                nbuf.at[pl.ds(lax.rem(ci, 2) * (8 * _FEAT), kb * _FEAT)],
                nsem.at[lax.rem(ci, 2)])
            if start:
                cp.start()
            else:
                cp.wait()

        def p0_out(ci, start, l=l, blk0=blk0, grow0=grow0, kb=kb):
            cp = pltpu.make_async_copy(
                tbuf.at[pl.ds(lax.rem(ci, 2) * 1024, kb * 128)],
                tab_hbm.at[pl.ds(grow0 + (blk0 + ci * kb) * 128, kb * 128)],
                tsem.at[lax.rem(ci, 2)])
            if start:
                cp.start()
            else:
                cp.wait()

        p0_in(jnp.int32(0), True)

        def p0_body(ci, carry, kb=kb, ni=ni):
            s = lax.rem(ci, 2)

            @pl.when(ci + 1 < ni)
            def _pref():
                p0_in(ci + 1, True)

            @pl.when(ci >= 2)
            def _drain():
                p0_out(ci - 2, False)

            p0_in(ci, False)
            nrow0 = tp_col + s * (8 * _FEAT)
            trow0v = tp_row + s * 1024

            @plsc.parallel_loop(0, kb * 64, 1, unroll=16)
            def _relane(k):
                rowv = nrow0 + lax.shift_right_logical(k, 6) * _FEAT
                colv = tp_row + 2 * lax.rem(k, 64)
                v = plsc.load_gather(nbuf, [rowv, colv])
                plsc.store_scatter(tbuf, [trow0v + 2 * k, tp_col], v)

            p0_out(ci, True)
            return carry

        lax.fori_loop(0, ni, p0_body, jnp.int32(0))
        if ni >= 2:
            p0_out(jnp.int32(ni - 2), False)
        p0_out(jnp.int32(ni - 1), False)

    plsc.subcore_barrier()

    base = wid * _CHUNK
    npts_w = jnp.minimum(jnp.int32(_CHUNK), jnp.int32(_NPTS) - base)
    nb = lax.shift_right_logical(npts_w, 4)
    ng = lax.shift_right_logical(npts_w, 6)

    pltpu.sync_copy(pts_hbm.at[0, pl.ds(base, _CHUNK)], xv)
    pltpu.sync_copy(pts_hbm.at[1, pl.ds(base, _CHUNK)], yv)
    pltpu.sync_copy(pts_hbm.at[2, pl.ds(base, _CHUNK)], zv)

    def coords(b):
        row0 = b * _B
        x = xv[pl.ds(row0, _B)]
        y = yv[pl.ds(row0, _B)]
        z = zv[pl.ds(row0, _B)]
        x = jnp.minimum(jnp.maximum(x, 0.0), 1.0)
        y = jnp.minimum(jnp.maximum(y, 0.0), 1.0)
        z = jnp.minimum(jnp.maximum(z, 0.0), 1.0)
        return x, y, z

    def lod_setup(x, y, z, res):
        scale = jnp.float32(res - 1)
        cx = x * scale
        cy = y * scale
        cz = z * scale
        ix = jnp.minimum(cx.astype(jnp.int32), res - 2)
        iy = jnp.minimum(cy.astype(jnp.int32), res - 2)
        iz = jnp.minimum(cz.astype(jnp.int32), res - 2)
        f000 = (ix * res + iy) * res + iz
        return (cx - ix.astype(jnp.float32), cy - iy.astype(jnp.float32),
                cz - iz.astype(jnp.float32), f000)

    def fire(b):
        slot = lax.rem(b, _NSLOT)
        x, y, z = coords(b)
        for l, res in enumerate(_LODS):
            if l == 0:
                continue
            _, _, _, f000 = lod_setup(x, y, z, res)
            f000 = f000 + _GOFF[l]
            offs = (0, 1, res, res + 1,
                    res * res, res * res + 1, res * res + res, res * res + res + 1)
            for c, off in enumerate(offs):
                idx_v[slot, l, pl.ds(c * _B, _B)] = f000 + off
            pltpu.async_copy(tab_hbm.at[idx_v.at[slot, l]],
                             buf_v.at[pl.ds((slot * 4 + l) * (8 * _B), 8 * _B)],
                             gsem.at[slot])

    def wait_gathers(slot):
        for l in range(1, 4):
            pltpu.make_async_copy(
                tab_hbm.at[idx_v.at[slot, l]],
                buf_v.at[pl.ds((slot * 4 + l) * (8 * _B), 8 * _B)],
                gsem.at[slot]).wait()

    cb0 = wid * (_CHUNK // 128)

    def oflush_copies(g, width):
        p = lax.rem(g, 2)
        cb = cb0 + g
        return [pltpu.make_async_copy(
                    oblk.at[pl.ds(p * 32 + fg * _FEAT, _FEAT), pl.ds(0, width)],
                    out_hbm.at[pl.ds((fg * _NCB + cb) * _FEAT, _FEAT),
                               pl.ds(0, width)],
                    osem.at[p])
                for fg in range(4)]

    fire(jnp.int32(0))
    fire(jnp.int32(1))
    fire(jnp.int32(2))

    def batch_body(b, carry):
        slot = lax.rem(b, _NSLOT)
        g = lax.shift_right_logical(b, 3)
        bo = lax.rem(b, 8)

        @pl.when(b + 3 < nb)
        def _prefetch():
            fire(b + 3)

        @pl.when(jnp.logical_and(bo == 0, g >= 2))
        def _drain_out():
            for cp in oflush_copies(g - 2, 128):
                cp.wait()

        wait_gathers(slot)

        x, y, z = coords(b)
        par32 = lax.rem(g, 2) * 32
        lanev = lax.rem(b, 8) * _B + iota
        for l, res in enumerate(_LODS):
            fx, fy, fz, f000 = lod_setup(x, y, z, res)
            gx = 1.0 - fx
            gy = 1.0 - fy
            gz = 1.0 - fz
            u00 = gx * gy
            u01 = gx * fy
            u10 = fx * gy
            u11 = fx * fy
            w = (u00 * gz, u00 * fz, u01 * gz, u01 * fz,
                 u10 * gz, u10 * fz, u11 * gz, u11 * fz)
            orow0 = jnp.full((_B,), par32 + l * _FEAT, jnp.int32)
            if l == 0:
                offs = (0, 1, res, res + 1, res * res, res * res + 1,
                        res * res + res, res * res + res + 1)

                @plsc.parallel_loop(0, _FEAT, 1, unroll=8)
                def _blend0_f(f):
                    col = jnp.full((_B,), 0, jnp.int32) + f
                    acc = w[0] * plsc.load_gather(g0tile, [f000, col])
                    for c in range(1, 8):
                        v = plsc.load_gather(g0tile, [f000 + offs[c], col])
                        acc = acc + w[c] * v
                    plsc.store_scatter(oblk, [orow0 + f, lanev], acc)

                continue
            rb = (slot * 4 + l) * (8 * _B) + iota

            @plsc.parallel_loop(0, _FEAT, 1, unroll=8)
            def _blend_f(f):
                col = jnp.full((_B,), 0, jnp.int32) + f
                acc = w[0] * plsc.load_gather(buf_v, [rb, col])
                for c in range(1, 8):
                    v = plsc.load_gather(buf_v, [rb + c * _B, col])
                    acc = acc + w[c] * v
                plsc.store_scatter(oblk, [orow0 + f, lanev], acc)

        @pl.when(bo == 7)
        def _flush():
            for cp in oflush_copies(g, 128):
                cp.start()

        return carry

    lax.fori_loop(0, nb, batch_body, jnp.int32(0))

    ngf = lax.shift_right_logical(nb, 3)

    @pl.when(lax.rem(nb, 8) != 0)
    def _partial_flush():
        for cp in oflush_copies(ngf, 64):
            cp.start()

    @pl.when(lax.rem(nb, 8) == 0)
    def _drain_m2():
        for cp in oflush_copies(ngf - 2, 128):
            cp.wait()

    for cp in oflush_copies(ngf - 1, 128):
        cp.wait()

    @pl.when(lax.rem(nb, 8) != 0)
    def _partial_drain():
        for cp in oflush_copies(ngf, 64):
            cp.wait()


def kernel(pts, grid0, grid1, grid2, grid3):
    ptst = jnp.pad(pts.T, ((0, 0), (0, _NPAD - _NPTS)))
    nats = [g.reshape(r ** 3 // 128, 128, _FEAT).transpose(0, 2, 1)
            .reshape(r ** 3 // 128 * _FEAT, 128)
            for g, r in zip((grid0, grid1, grid2, grid3), _LODS)]
    mesh = plsc.VectorSubcoreMesh(core_axis_name="c", subcore_axis_name="s")
    k = functools.partial(
        pl.kernel,
        mesh=mesh,
        out_type=(jax.ShapeDtypeStruct((4 * _NCB * _FEAT, 128), jnp.float32),
                  jax.ShapeDtypeStruct((_TROWS, _FEAT), jnp.float32)),
        compiler_params=pltpu.CompilerParams(
            needs_layout_passes=False, use_tc_tiling_on_sc=False),
        scratch_types=(
            [pltpu.VMEM((_CHUNK,), jnp.float32) for _ in range(3)]
            + [pltpu.VMEM((_NSLOT, 4, 8 * _B), jnp.int32),
               pltpu.VMEM((_NSLOT * 4 * 8 * _B, _FEAT), jnp.float32),
               pltpu.VMEM((2 * _OUTW, 128), jnp.float32),
               pltpu.VMEM((2 * 8 * _FEAT, 128), jnp.float32),
               pltpu.VMEM((2 * 8 * 128, _FEAT), jnp.float32),
               pltpu.VMEM((16 ** 3, _FEAT), jnp.float32),
               pltpu.SemaphoreType.DMA((_NSLOT,)),
               pltpu.SemaphoreType.DMA((2,)),
               pltpu.SemaphoreType.DMA((2,)),
               pltpu.SemaphoreType.DMA((2,))]
        ),
    )(_interp_kernel)
    out4, _ = k(ptst, *nats)
    out4 = out4.reshape(4, _NCB, _FEAT, 128)
    return out4.transpose(1, 3, 0, 2).reshape(_NCB * 128, _OUTW)[:_NPTS]


# speedup vs baseline: 17.1054x; 1.0000x over previous
import functools

import jax
import jax.numpy as jnp
from jax import lax
from jax.experimental import pallas as pl
from jax.experimental.pallas import tpu as pltpu
from jax.experimental.pallas import tpu_sc as plsc

_LODS = (16, 32, 64, 128)
_FEAT = 8
_NPTS = 200000
_NC = 2
_NS = 16
_NW = _NC * _NS
_B = 16
_GRP = 4
_NSLOT = 6
_OUTW = 4 * _FEAT
_GOFF = (0, 16 ** 3, 16 ** 3 + 32 ** 3, 16 ** 3 + 32 ** 3 + 64 ** 3)
_TROWS = 16 ** 3 + 32 ** 3 + 64 ** 3 + 128 ** 3
_NBLK = tuple(r ** 3 // 128 for r in _LODS)
_KB = (2, 8, 8, 8)
_CHUNK = 6272
_NPAD = _NW * _CHUNK
_NCB = (_NPTS + 127) // 128


def _interp_kernel(pts_hbm, g0, g1, g2, g3, out_hbm, tab_hbm,
                   xv, yv, zv, idx_v, buf_v, oblk, nbuf, tbuf, g0tile,
                   gsem, osem, nsem, tsem):
    grids = (g0, g1, g2, g3)
    cid = lax.axis_index("c")
    sid = lax.axis_index("s")
    wid = sid * _NC + cid
    iota = lax.iota(jnp.int32, _B)

    tp_row = iota >> 3
    tp_col = iota & 7

    def g0_in(ci, start):
        cp = pltpu.make_async_copy(
            g0.at[pl.ds(ci * 64, 64)],
            nbuf.at[pl.ds(lax.rem(ci, 2) * 64, 64)],
            nsem.at[lax.rem(ci, 2)])
        if start:
            cp.start()
        else:
            cp.wait()

    g0_in(jnp.int32(0), True)

    def g0_body(ci, carry):
        s = lax.rem(ci, 2)

        @pl.when(ci + 1 < 4)
        def _pref():
            g0_in(ci + 1, True)

        g0_in(ci, False)
        nrow0 = tp_col + s * 64
        trow0v = tp_row + ci * 1024

        @plsc.parallel_loop(0, 512, 1, unroll=16)
        def _relane0(k):
            rowv = nrow0 + lax.shift_right_logical(k, 6) * _FEAT
            colv = tp_row + 2 * lax.rem(k, 64)
            v = plsc.load_gather(nbuf, [rowv, colv])
            plsc.store_scatter(g0tile, [trow0v + 2 * k, tp_col], v)

        return carry

    lax.fori_loop(0, 4, g0_body, jnp.int32(0))

    for l in range(1, 4):
        nbt = _NBLK[l] // _NS
        kb = _KB[l]
        ni = nbt // kb
        blk0 = sid * nbt
        grow0 = _GOFF[l]

        def p0_in(ci, start, l=l, blk0=blk0, kb=kb):
            cp = pltpu.make_async_copy(
                grids[l].at[pl.ds((blk0 + ci * kb) * _FEAT, kb * _FEAT)],
                nbuf.at[pl.ds(lax.rem(ci, 2) * (8 * _FEAT), kb * _FEAT)],
                nsem.at[lax.rem(ci, 2)])
            if start:
                cp.start()
            else:
                cp.wait()

        def p0_out(ci, start, l=l, blk0=blk0, grow0=grow0, kb=kb):
            cp = pltpu.make_async_copy(
                tbuf.at[pl.ds(lax.rem(ci, 2) * 1024, kb * 128)],
                tab_hbm.at[pl.ds(grow0 + (blk0 + ci * kb) * 128, kb * 128)],
                tsem.at[lax.rem(ci, 2)])
            if start:
                cp.start()
            else:
                cp.wait()

        p0_in(jnp.int32(0), True)

        def p0_body(ci, carry, kb=kb, ni=ni):
            s = lax.rem(ci, 2)

            @pl.when(ci + 1 < ni)
            def _pref():
                p0_in(ci + 1, True)

            @pl.when(ci >= 2)
            def _drain():
                p0_out(ci - 2, False)

            p0_in(ci, False)
            nrow0 = tp_col + s * (8 * _FEAT)
            trow0v = tp_row + s * 1024

            @plsc.parallel_loop(0, kb * 64, 1, unroll=16)
            def _relane(k):
                rowv = nrow0 + lax.shift_right_logical(k, 6) * _FEAT
                colv = tp_row + 2 * lax.rem(k, 64)
                v = plsc.load_gather(nbuf, [rowv, colv])
                plsc.store_scatter(tbuf, [trow0v + 2 * k, tp_col], v)

            p0_out(ci, True)
            return carry

        lax.fori_loop(0, ni, p0_body, jnp.int32(0))
        if ni >= 2:
            p0_out(jnp.int32(ni - 2), False)
        p0_out(jnp.int32(ni - 1), False)

    plsc.subcore_barrier()

    base = wid * _CHUNK
    npts_w = jnp.minimum(jnp.int32(_CHUNK), jnp.int32(_NPTS) - base)
    nb = lax.shift_right_logical(npts_w, 4)
    ng = lax.shift_right_logical(npts_w, 6)

    pltpu.sync_copy(pts_hbm.at[0, pl.ds(base, _CHUNK)], xv)
    pltpu.sync_copy(pts_hbm.at[1, pl.ds(base, _CHUNK)], yv)
    pltpu.sync_copy(pts_hbm.at[2, pl.ds(base, _CHUNK)], zv)

    def coords(b):
        row0 = b * _B
        x = xv[pl.ds(row0, _B)]
        y = yv[pl.ds(row0, _B)]
        z = zv[pl.ds(row0, _B)]
        x = jnp.minimum(jnp.maximum(x, 0.0), 1.0)
        y = jnp.minimum(jnp.maximum(y, 0.0), 1.0)
        z = jnp.minimum(jnp.maximum(z, 0.0), 1.0)
        return x, y, z

    def lod_setup(x, y, z, res):
        scale = jnp.float32(res - 1)
        cx = x * scale
        cy = y * scale
        cz = z * scale
        ix = jnp.minimum(cx.astype(jnp.int32), res - 2)
        iy = jnp.minimum(cy.astype(jnp.int32), res - 2)
        iz = jnp.minimum(cz.astype(jnp.int32), res - 2)
        f000 = (ix * res + iy) * res + iz
        return (cx - ix.astype(jnp.float32), cy - iy.astype(jnp.float32),
                cz - iz.astype(jnp.float32), f000)

    def fire(b):
        slot = lax.rem(b, _NSLOT)
        x, y, z = coords(b)
        for l, res in enumerate(_LODS):
            if l == 0:
                continue
            _, _, _, f000 = lod_setup(x, y, z, res)
            f000 = f000 + _GOFF[l]
            offs = (0, 1, res, res + 1,
                    res * res, res * res + 1, res * res + res, res * res + res + 1)
            for c, off in enumerate(offs):
                idx_v[slot, l, pl.ds(c * _B, _B)] = f000 + off
            pltpu.async_copy(tab_hbm.at[idx_v.at[slot, l]],
                             buf_v.at[pl.ds((slot * 4 + l) * (8 * _B), 8 * _B)],
                             gsem.at[slot])

    def wait_gathers(slot):
        for l in range(1, 4):
            pltpu.make_async_copy(
                tab_hbm.at[idx_v.at[slot, l]],
                buf_v.at[pl.ds((slot * 4 + l) * (8 * _B), 8 * _B)],
                gsem.at[slot]).wait()

    cb0 = wid * (_CHUNK // 128)

    def oflush_copies(g, width):
        p = lax.rem(g, 2)
        cb = cb0 + g
        return [pltpu.make_async_copy(
                    oblk.at[pl.ds(p * 32 + fg * _FEAT, _FEAT), pl.ds(0, width)],
                    out_hbm.at[pl.ds((fg * _NCB + cb) * _FEAT, _FEAT),
                               pl.ds(0, width)],
                    osem.at[p])
                for fg in range(4)]

    for i in range(_NSLOT - 1):
        fire(jnp.int32(i))

    def batch_body(b, carry):
        slot = lax.rem(b, _NSLOT)
        g = lax.shift_right_logical(b, 3)
        bo = lax.rem(b, 8)

        @pl.when(b + (_NSLOT - 1) < nb)
        def _prefetch():
            fire(b + (_NSLOT - 1))

        @pl.when(jnp.logical_and(bo == 0, g >= 2))
        def _drain_out():
            for cp in oflush_copies(g - 2, 128):
                cp.wait()

        wait_gathers(slot)

        x, y, z = coords(b)
        par32 = lax.rem(g, 2) * 32
        lanev = lax.rem(b, 8) * _B + iota
        for l, res in enumerate(_LODS):
            fx, fy, fz, f000 = lod_setup(x, y, z, res)
            gx = 1.0 - fx
            gy = 1.0 - fy
            gz = 1.0 - fz
            u00 = gx * gy
            u01 = gx * fy
            u10 = fx * gy
            u11 = fx * fy
            w = (u00 * gz, u00 * fz, u01 * gz, u01 * fz,
                 u10 * gz, u10 * fz, u11 * gz, u11 * fz)
            orow0 = jnp.full((_B,), par32 + l * _FEAT, jnp.int32)
            if l == 0:
                offs = (0, 1, res, res + 1, res * res, res * res + 1,
                        res * res + res, res * res + res + 1)

                @plsc.parallel_loop(0, _FEAT, 1, unroll=8)
                def _blend0_f(f):
                    col = jnp.full((_B,), 0, jnp.int32) + f
                    acc = w[0] * plsc.load_gather(g0tile, [f000, col])
                    for c in range(1, 8):
                        v = plsc.load_gather(g0tile, [f000 + offs[c], col])
                        acc = acc + w[c] * v
                    plsc.store_scatter(oblk, [orow0 + f, lanev], acc)

                continue
            rb = (slot * 4 + l) * (8 * _B) + iota

            @plsc.parallel_loop(0, _FEAT, 1, unroll=8)
            def _blend_f(f):
                col = jnp.full((_B,), 0, jnp.int32) + f
                acc = w[0] * plsc.load_gather(buf_v, [rb, col])
                for c in range(1, 8):
                    v = plsc.load_gather(buf_v, [rb + c * _B, col])
                    acc = acc + w[c] * v
                plsc.store_scatter(oblk, [orow0 + f, lanev], acc)

        @pl.when(bo == 7)
        def _flush():
            for cp in oflush_copies(g, 128):
                cp.start()

        return carry

    lax.fori_loop(0, nb, batch_body, jnp.int32(0))

    ngf = lax.shift_right_logical(nb, 3)

    @pl.when(lax.rem(nb, 8) != 0)
    def _partial_flush():
        for cp in oflush_copies(ngf, 64):
            cp.start()

    @pl.when(lax.rem(nb, 8) == 0)
    def _drain_m2():
        for cp in oflush_copies(ngf - 2, 128):
            cp.wait()

    for cp in oflush_copies(ngf - 1, 128):
        cp.wait()

    @pl.when(lax.rem(nb, 8) != 0)
    def _partial_drain():
        for cp in oflush_copies(ngf, 64):
            cp.wait()


def kernel(pts, grid0, grid1, grid2, grid3):
    ptst = jnp.pad(pts.T, ((0, 0), (0, _NPAD - _NPTS)))
    nats = [g.reshape(r ** 3 // 128, 128, _FEAT).transpose(0, 2, 1)
            .reshape(r ** 3 // 128 * _FEAT, 128)
            for g, r in zip((grid0, grid1, grid2, grid3), _LODS)]
    mesh = plsc.VectorSubcoreMesh(core_axis_name="c", subcore_axis_name="s")
    k = functools.partial(
        pl.kernel,
        mesh=mesh,
        out_type=(jax.ShapeDtypeStruct((4 * _NCB * _FEAT, 128), jnp.float32),
                  jax.ShapeDtypeStruct((_TROWS, _FEAT), jnp.float32)),
        compiler_params=pltpu.CompilerParams(
            needs_layout_passes=False, use_tc_tiling_on_sc=False),
        scratch_types=(
            [pltpu.VMEM((_CHUNK,), jnp.float32) for _ in range(3)]
            + [pltpu.VMEM((_NSLOT, 4, 8 * _B), jnp.int32),
               pltpu.VMEM((_NSLOT * 4 * 8 * _B, _FEAT), jnp.float32),
               pltpu.VMEM((2 * _OUTW, 128), jnp.float32),
               pltpu.VMEM((2 * 8 * _FEAT, 128), jnp.float32),
               pltpu.VMEM((2 * 8 * 128, _FEAT), jnp.float32),
               pltpu.VMEM((16 ** 3, _FEAT), jnp.float32),
               pltpu.SemaphoreType.DMA((_NSLOT,)),
               pltpu.SemaphoreType.DMA((2,)),
               pltpu.SemaphoreType.DMA((2,)),
               pltpu.SemaphoreType.DMA((2,))]
        ),
    )(_interp_kernel)
    out4, _ = k(ptst, *nats)
    out4 = out4.reshape(4, _NCB, _FEAT, 128)
    return out4.transpose(1, 3, 0, 2).reshape(_NCB * 128, _OUTW)[:_NPTS]
